# baseline (speedup 1.0000x reference)
import functools

import jax
import jax.numpy as jnp
from jax import lax
from jax.experimental import pallas as pl
from jax.experimental.pallas import tpu as pltpu
from jax.experimental.pallas import tpu_sc as plsc

NC = 2
NS = 16
L = 16
NW = NC * NS
NB = 4


def _make_sc_kernel(batch, maxlen, embed_dim):
    seq_per_w = batch // NW
    assert batch % NW == 0 and seq_per_w % NB == 0 and seq_per_w >= 2 * NB
    vregs_per_row = embed_dim // L

    mesh = plsc.VectorSubcoreMesh(core_axis_name="c", subcore_axis_name="s")

    @functools.partial(
        pl.kernel,
        out_type=jax.ShapeDtypeStruct((batch, maxlen, embed_dim), jnp.float32),
        mesh=mesh,
        scratch_types=[
            pltpu.VMEM((seq_per_w, maxlen), jnp.int32),
            [pltpu.VMEM((maxlen, embed_dim), jnp.float32) for _ in range(NB)],
            pltpu.VMEM((maxlen, embed_dim), jnp.float32),
            [pltpu.SemaphoreType.DMA for _ in range(NB)],
            [pltpu.SemaphoreType.DMA for _ in range(NB)],
        ],
        compiler_params=pltpu.CompilerParams(use_tc_tiling_on_sc=False),
    )
    def k(x_hbm, tok_hbm, pos_hbm, out_hbm, idx_all, rows, pos_v, semg, sems):
        wid = lax.axis_index("s") * NC + lax.axis_index("c")
        wseq = wid * seq_per_w

        pltpu.sync_copy(pos_hbm, pos_v)
        pltpu.sync_copy(x_hbm.at[pl.ds(wseq, seq_per_w)], idx_all)

        def issue_gather(cur, b):
            pltpu.async_copy(tok_hbm.at[idx_all.at[cur]], rows[b], semg[b])

        def wait_gather(cur, b):
            pltpu.make_async_copy(
                tok_hbm.at[idx_all.at[cur]], rows[b], semg[b]).wait()

        def issue_store(cur, b):
            pltpu.async_copy(rows[b], out_hbm.at[wseq + cur], sems[b])

        def wait_store(cur, b):
            pltpu.make_async_copy(
                rows[b], out_hbm.at[wseq + cur], sems[b]).wait()

        def add_pos(rb):
            @pl.loop(0, maxlen, unroll=4)
            def _tok_loop(t):
                for v in range(vregs_per_row):
                    sl = pl.ds(v * L, L)
                    rb[t, sl] = rb[t, sl] + pos_v[t, sl]

        def emit(cur, b, wait_prev, issue_next):
            wait_gather(cur, b)
            add_pos(rows[b])
            issue_store(cur, b)
            br = (b + NB - 1) % NB
            if wait_prev:
                wait_store(cur - 1, br)
            if issue_next:
                issue_gather(cur + NB - 1, br)

        for c in range(NB - 1):
            issue_gather(c, c)

        for c in range(NB):
            emit(c, c, wait_prev=(c >= 1), issue_next=True)

        @pl.loop(NB, seq_per_w - NB, step=NB)
        def _seq_loop(ci):
            for j in range(NB):
                emit(ci + j, j, wait_prev=True, issue_next=True)

        for c in range(seq_per_w - NB, seq_per_w):
            go = c + NB - 1 < seq_per_w
            emit(c, c % NB, wait_prev=go, issue_next=go)

        for c in range(seq_per_w - NB, seq_per_w):
            wait_store(c, c % NB)

    return k


def kernel(x, token_table, pos_table):
    batch, maxlen = x.shape
    vocab, embed_dim = token_table.shape
    k = _make_sc_kernel(batch, maxlen, embed_dim)
    return k(x.astype(jnp.int32), token_table, pos_table)
